# baseline (speedup 1.0000x reference)
import functools

import jax
import jax.numpy as jnp
from jax import lax
from jax.experimental import pallas as pl
from jax.experimental.pallas import tpu as pltpu
from jax.experimental.pallas import tpu_sc as plsc

_B, _C, _H, _W = 8, 2, 512, 512
_N = _B * _H * _W
_NW = 32
_WPB = _NW // _B
_RPW = _H // _WPB
_CHR = 32
_NCH = _RPW // _CHR
_L = 16

_Q = (0.9999970510848344, -0.4998254028857509, 0.33078744547883354,
      -0.234172411224585, 0.14810505362112691, -0.06576904117079967,
      0.014026606298625958)


def _bce(xv, tf):
    ax = jnp.abs(xv)
    e = jnp.exp(-ax)
    q = jnp.float32(_Q[6]) * e + jnp.float32(_Q[5])
    for c in _Q[4::-1]:
        q = q * e + jnp.float32(c)
    return jnp.maximum(xv, jnp.float32(0.0)) + e * q - tf * xv


def _sc_reduce_body(x_hbm, t_hbm, out_hbm, xb0, tb0, xb1, tb1, accb,
                    sem0, sem1):
    wid = lax.axis_index("s") * 2 + lax.axis_index("c")
    b = wid // _WPB
    r0 = (wid % _WPB) * _RPW
    xbufs, tbufs, sems = (xb0, xb1), (tb0, tb1), (sem0, sem1)

    def start(c):
        s = sems[c & 1]
        cx = pltpu.async_copy(
            x_hbm.at[2 * b + 1, pl.ds(r0 + c * _CHR, _CHR), :], xbufs[c & 1], s)
        ct = pltpu.async_copy(
            t_hbm.at[b, pl.ds(r0 + c * _CHR, _CHR), :], tbufs[c & 1], s)
        return cx, ct

    acc = (jnp.zeros((_L,), jnp.float32),) * 3
    cps = start(0)
    for c in range(_NCH):
        nxt = start(c + 1) if c + 1 < _NCH else None
        cps[0].wait()
        cps[1].wait()
        xb, tb = xbufs[c & 1], tbufs[c & 1]

        def row_body(r, a, xb=xb, tb=tb):
            s_all, s_pos, c_pos = a
            for j in range(_W // _L):
                xv = xb[r, pl.ds(j * _L, _L)]
                tf = tb[r, pl.ds(j * _L, _L)].astype(jnp.float32)
                bce = _bce(xv, tf)
                s_all = s_all + bce
                s_pos = s_pos + tf * bce
                c_pos = c_pos + tf
            return (s_all, s_pos, c_pos)

        acc = lax.fori_loop(0, _CHR, row_body, acc)
        if nxt is not None:
            cps = nxt

    s_all, s_pos, c_pos = acc
    accb[pl.ds(0, _L)] = s_all
    accb[pl.ds(_L, _L)] = s_pos
    accb[pl.ds(2 * _L, _L)] = c_pos
    for i in range(3):
        pltpu.sync_copy(accb.at[pl.ds(i * _L, _L)],
                        out_hbm.at[i, pl.ds(wid * _L, _L)])


@functools.lru_cache(maxsize=None)
def _sc_reduce():
    return pl.kernel(
        _sc_reduce_body,
        out_type=jax.ShapeDtypeStruct((3, _NW * _L), jnp.float32),
        mesh=plsc.VectorSubcoreMesh(core_axis_name="c", subcore_axis_name="s"),
        scratch_types=[
            pltpu.VMEM((_CHR, _W), jnp.float32),
            pltpu.VMEM((_CHR, _W), jnp.int32),
            pltpu.VMEM((_CHR, _W), jnp.float32),
            pltpu.VMEM((_CHR, _W), jnp.int32),
            pltpu.VMEM((48,), jnp.float32),
            pltpu.SemaphoreType.DMA,
            pltpu.SemaphoreType.DMA,
        ],
        compiler_params=pltpu.CompilerParams(use_tc_tiling_on_sc=True),
    )


def _finish_body(parts_ref, x_hbm, t_hbm, out_ref, xb, tb, keys, sem):
    s_all = jnp.sum(parts_ref[0, :])
    s_pos = jnp.sum(parts_ref[1, :])
    n_pos_f = jnp.sum(parts_ref[2, :])

    num_pos = n_pos_f.astype(jnp.int32)
    num_neg = jnp.int32(_N) - num_pos
    k_pos = jnp.minimum(num_neg, 20 * num_pos)
    k_empty = jnp.maximum(
        1, (num_neg.astype(jnp.float32) * jnp.float32(0.01)).astype(jnp.int32))
    k = jnp.where(num_pos > 0, k_pos, k_empty)
    kf = jnp.maximum(k, 1).astype(jnp.float32)
    pos_keep = jnp.where(num_pos > 0, s_pos / jnp.maximum(n_pos_f, 1.0), 0.0)

    neg_fast = jnp.where(num_neg > 0, (s_all - s_pos) / kf, 0.0)
    out_ref[0] = pos_keep + neg_fast

    @pl.when(k != num_neg)
    def _():
        def plane(b, carry):
            cpx = pltpu.make_async_copy(x_hbm.at[2 * b + 1], xb, sem)
            cpx.start()
            cpx.wait()
            cpt = pltpu.make_async_copy(t_hbm.at[b], tb, sem)
            cpt.start()
            cpt.wait()
            xv = xb[...]
            tv = tb[...]
            tf = tv.astype(jnp.float32)
            bce = (jnp.maximum(xv, 0.0) + jnp.log1p(jnp.exp(-jnp.abs(xv)))
                   - tf * xv)
            keys[pl.ds(b * _H, _H), :] = jnp.where(
                tv > 0, jnp.int32(0), pltpu.bitcast(bce, jnp.int32))
            return carry

        lax.fori_loop(0, _B, plane, 0)

        def bit_body(i, prefix):
            cand = prefix | jnp.left_shift(jnp.int32(1), 30 - i)
            cnt = jnp.sum((keys[...] >= cand).astype(jnp.int32))
            return jnp.where(cnt >= k, cand, prefix)

        prefix = lax.fori_loop(0, 31, bit_body, jnp.int32(0))
        kv = keys[...]
        vals = pltpu.bitcast(kv, jnp.float32)
        gt = kv > prefix
        cnt_gt = jnp.sum(gt.astype(jnp.int32))
        sum_gt = jnp.sum(jnp.where(gt, vals, 0.0))
        thr = jnp.max(jnp.where(kv == prefix, vals, 0.0))
        topk_sum = sum_gt + (k - cnt_gt).astype(jnp.float32) * thr
        out_ref[0] = pos_keep + topk_sum / kf


_finish_call = pl.pallas_call(
    _finish_body,
    in_specs=[
        pl.BlockSpec(memory_space=pltpu.VMEM),
        pl.BlockSpec(memory_space=pltpu.MemorySpace.HBM),
        pl.BlockSpec(memory_space=pltpu.MemorySpace.HBM),
    ],
    out_specs=pl.BlockSpec(memory_space=pltpu.SMEM),
    out_shape=jax.ShapeDtypeStruct((1,), jnp.float32),
    scratch_shapes=[
        pltpu.VMEM((_H, _W), jnp.float32),
        pltpu.VMEM((_H, _W), jnp.int32),
        pltpu.VMEM((_B * _H, _W), jnp.int32),
        pltpu.SemaphoreType.DMA,
    ],
)


def kernel(logits, target):
    x3 = logits.reshape(_B * _C, _H, _W)
    parts = _sc_reduce()(x3, target)
    return _finish_call(parts, x3, target)[0]


# speedup vs baseline: 1.2933x; 1.2933x over previous
import functools

import jax
import jax.numpy as jnp
from jax import lax
from jax.experimental import pallas as pl
from jax.experimental.pallas import tpu as pltpu
from jax.experimental.pallas import tpu_sc as plsc

_B, _C, _H, _W = 8, 2, 512, 512
_N = _B * _H * _W
_NW = 32
_WPB = _NW // _B
_RPW = _H // _WPB
_CHR = 32
_NCH = _RPW // _CHR
_L = 16

_Q = (0.6930990815162659, -0.4991752803325653, 0.12153831124305725,
      0.006159262731671333, -0.010860699228942394, 0.0028300986159592867,
      -0.0003471552045084536, 1.5457020708709024e-05, 1.0346333283450804e-06,
      -1.4905096179518296e-07, 4.98571006701809e-09)


def _bce(xv, tf):
    ac = jnp.minimum(jnp.abs(xv), jnp.float32(8.0))
    h = jnp.float32(_Q[10]) * ac + jnp.float32(_Q[9])
    for c in _Q[8::-1]:
        h = h * ac + jnp.float32(c)
    h = jnp.maximum(h, jnp.float32(0.0))
    return jnp.maximum(xv, jnp.float32(0.0)) + h - tf * xv


_UNROLL = 4


def _sc_reduce_body(x_hbm, t_hbm, out_hbm, xb0, tb0, xb1, tb1,
                    sa_b, sp_b, cp_b, sem0, sem1):
    wid = lax.axis_index("s") * 2 + lax.axis_index("c")
    b = wid // _WPB
    r0 = (wid % _WPB) * _RPW
    xbufs, tbufs, sems = (xb0, xb1), (tb0, tb1), (sem0, sem1)

    def start(c):
        s = sems[c & 1]
        cx = pltpu.async_copy(
            x_hbm.at[2 * b + 1, pl.ds(r0 + c * _CHR, _CHR), :], xbufs[c & 1], s)
        ct = pltpu.async_copy(
            t_hbm.at[b, pl.ds(r0 + c * _CHR, _CHR), :], tbufs[c & 1], s)
        return cx, ct

    z = jnp.zeros((_L,), jnp.float32)
    acc = (z,) * (3 * _UNROLL)

    cps = start(0)
    for c in range(_NCH):
        nxt = start(c + 1) if c + 1 < _NCH else None
        cps[0].wait()
        cps[1].wait()
        xb, tb = xbufs[c & 1], tbufs[c & 1]

        def row_body(r, a, xb=xb, tb=tb):
            def grp_body(g, a2):
                a2 = list(a2)
                for u in range(_UNROLL):
                    sl = pl.ds(g * (_UNROLL * _L) + u * _L, _L)
                    xv = xb[r, sl]
                    tf = tb[r, sl].astype(jnp.float32)
                    bce = _bce(xv, tf)
                    a2[3 * u] = a2[3 * u] + bce
                    a2[3 * u + 1] = a2[3 * u + 1] + tf * bce
                    a2[3 * u + 2] = a2[3 * u + 2] + tf
                return tuple(a2)

            return lax.fori_loop(0, _W // (_UNROLL * _L), grp_body, a)

        acc = lax.fori_loop(0, _CHR, row_body, acc)
        if nxt is not None:
            cps = nxt

    sa_b[...] = sum(acc[0::3][1:], acc[0])
    sp_b[...] = sum(acc[1::3][1:], acc[1])
    cp_b[...] = sum(acc[2::3][1:], acc[2])
    for i, buf in enumerate((sa_b, sp_b, cp_b)):
        pltpu.sync_copy(buf, out_hbm.at[i, pl.ds(wid * _L, _L)])


@functools.lru_cache(maxsize=None)
def _sc_reduce():
    return pl.kernel(
        _sc_reduce_body,
        out_type=jax.ShapeDtypeStruct((3, _NW * _L), jnp.float32),
        mesh=plsc.VectorSubcoreMesh(core_axis_name="c", subcore_axis_name="s"),
        scratch_types=[
            pltpu.VMEM((_CHR, _W), jnp.float32),
            pltpu.VMEM((_CHR, _W), jnp.int32),
            pltpu.VMEM((_CHR, _W), jnp.float32),
            pltpu.VMEM((_CHR, _W), jnp.int32),
            pltpu.VMEM((_L,), jnp.float32),
            pltpu.VMEM((_L,), jnp.float32),
            pltpu.VMEM((_L,), jnp.float32),
            pltpu.SemaphoreType.DMA,
            pltpu.SemaphoreType.DMA,
        ],
        compiler_params=pltpu.CompilerParams(use_tc_tiling_on_sc=True),
    )


def _finish_body(parts_ref, x_hbm, t_hbm, out_ref, xb, tb, keys, sem):
    s_all = jnp.sum(parts_ref[0, :])
    s_pos = jnp.sum(parts_ref[1, :])
    n_pos_f = jnp.sum(parts_ref[2, :])

    num_pos = n_pos_f.astype(jnp.int32)
    num_neg = jnp.int32(_N) - num_pos
    k_pos = jnp.minimum(num_neg, 20 * num_pos)
    k_empty = jnp.maximum(
        1, (num_neg.astype(jnp.float32) * jnp.float32(0.01)).astype(jnp.int32))
    k = jnp.where(num_pos > 0, k_pos, k_empty)
    kf = jnp.maximum(k, 1).astype(jnp.float32)
    pos_keep = jnp.where(num_pos > 0, s_pos / jnp.maximum(n_pos_f, 1.0), 0.0)

    neg_fast = jnp.where(num_neg > 0, (s_all - s_pos) / kf, 0.0)
    out_ref[0] = pos_keep + neg_fast

    @pl.when(k != num_neg)
    def _():
        def plane(b, carry):
            cpx = pltpu.make_async_copy(x_hbm.at[2 * b + 1], xb, sem)
            cpx.start()
            cpx.wait()
            cpt = pltpu.make_async_copy(t_hbm.at[b], tb, sem)
            cpt.start()
            cpt.wait()
            xv = xb[...]
            tv = tb[...]
            tf = tv.astype(jnp.float32)
            bce = (jnp.maximum(xv, 0.0) + jnp.log1p(jnp.exp(-jnp.abs(xv)))
                   - tf * xv)
            keys[pl.ds(b * _H, _H), :] = jnp.where(
                tv > 0, jnp.int32(0), pltpu.bitcast(bce, jnp.int32))
            return carry

        lax.fori_loop(0, _B, plane, 0)

        def bit_body(i, prefix):
            cand = prefix | jnp.left_shift(jnp.int32(1), 30 - i)
            cnt = jnp.sum((keys[...] >= cand).astype(jnp.int32))
            return jnp.where(cnt >= k, cand, prefix)

        prefix = lax.fori_loop(0, 31, bit_body, jnp.int32(0))
        kv = keys[...]
        vals = pltpu.bitcast(kv, jnp.float32)
        gt = kv > prefix
        cnt_gt = jnp.sum(gt.astype(jnp.int32))
        sum_gt = jnp.sum(jnp.where(gt, vals, 0.0))
        thr = jnp.max(jnp.where(kv == prefix, vals, 0.0))
        topk_sum = sum_gt + (k - cnt_gt).astype(jnp.float32) * thr
        out_ref[0] = pos_keep + topk_sum / kf


_finish_call = pl.pallas_call(
    _finish_body,
    in_specs=[
        pl.BlockSpec(memory_space=pltpu.VMEM),
        pl.BlockSpec(memory_space=pltpu.MemorySpace.HBM),
        pl.BlockSpec(memory_space=pltpu.MemorySpace.HBM),
    ],
    out_specs=pl.BlockSpec(memory_space=pltpu.SMEM),
    out_shape=jax.ShapeDtypeStruct((1,), jnp.float32),
    scratch_shapes=[
        pltpu.VMEM((_H, _W), jnp.float32),
        pltpu.VMEM((_H, _W), jnp.int32),
        pltpu.VMEM((_B * _H, _W), jnp.int32),
        pltpu.SemaphoreType.DMA,
    ],
)


def kernel(logits, target):
    x3 = logits.reshape(_B * _C, _H, _W)
    parts = _sc_reduce()(x3, target)
    return _finish_call(parts, x3, target)[0]
